# baseline (speedup 1.0000x reference)
import jax
import jax.numpy as jnp
from jax import lax
from jax.experimental import pallas as pl
from jax.experimental.pallas import tpu as pltpu
from jax.experimental.pallas import tpu_sc as plsc

D_VOCAB = 1_000_000
D_MODEL = 64
BATCH = 16384

NC = 2
NS = 16
LANES = 16
NW = NC * NS

RPW = BATCH // NW
RPC = 256
NCHUNK = RPW // RPC
IPC = 2 * RPC
ELEMS = IPC * D_MODEL
GROW = 128
NGROW = ELEMS // GROW
FIRE = 8


def _body(xf_hbm, wf_hbm, out_hbm, idx_v, offs_v, g_v, o_v, sem):
    wid = lax.axis_index("s") * NC + lax.axis_index("c")
    dvecs = [
        lax.iota(jnp.int32, LANES) * D_VOCAB + q * LANES * D_VOCAB
        for q in range(D_MODEL // LANES)
    ]

    for c in range(NCHUNK):
        idx_base = wid * (2 * RPW) + c * IPC
        pltpu.sync_copy(xf_hbm.at[pl.ds(idx_base, IPC)], idx_v)

        def build(v, carry):
            idx16 = idx_v[pl.ds(v * LANES, LANES)]
            for l in range(LANES):
                i = idx16[l]
                base = (v * LANES + l) * D_MODEL
                for q in range(D_MODEL // LANES):
                    offs_v[pl.ds(base + q * LANES, LANES)] = dvecs[q] + i
            return carry

        lax.fori_loop(0, IPC // LANES, build, 0)

        def gather_grp(gq, carry):
            base = gq * (FIRE * GROW)
            cps = []
            for r in range(FIRE):
                sl = pl.ds(base + r * GROW, GROW)
                cps.append(pltpu.async_copy(wf_hbm.at[offs_v.at[sl]], g_v.at[sl], sem))
            for cp in cps:
                cp.wait()
            return carry

        lax.fori_loop(0, NGROW // FIRE, gather_grp, 0)

        def psum(b, carry):
            gb = b * (2 * D_MODEL)
            ob = b * D_MODEL
            for q in range(D_MODEL // LANES):
                o_v[pl.ds(ob + q * LANES, LANES)] = (
                    g_v[pl.ds(gb + q * LANES, LANES)]
                    + g_v[pl.ds(gb + D_MODEL + q * LANES, LANES)]
                )
            return carry

        lax.fori_loop(0, RPC, psum, 0)

        out_base = (wid * RPW + c * RPC) * D_MODEL
        pltpu.sync_copy(o_v, out_hbm.at[pl.ds(out_base, RPC * D_MODEL)])


def _make_sc_call():
    mesh = plsc.VectorSubcoreMesh(
        core_axis_name="c", subcore_axis_name="s", num_cores=NC, num_subcores=NS
    )
    return pl.kernel(
        _body,
        out_type=jax.ShapeDtypeStruct((BATCH * D_MODEL,), jnp.float32),
        mesh=mesh,
        scratch_types=[
            pltpu.VMEM((IPC,), jnp.int32),
            pltpu.VMEM((ELEMS,), jnp.int32),
            pltpu.VMEM((ELEMS,), jnp.float32),
            pltpu.VMEM((RPC * D_MODEL,), jnp.float32),
            pltpu.SemaphoreType.DMA,
        ],
    )


def kernel(x, W_E):
    xf = x.reshape(-1).astype(jnp.int32)
    wf = W_E.reshape(-1)
    out = _make_sc_call()(xf, wf)
    return out.reshape(BATCH, 1, D_MODEL)


# speedup vs baseline: 1.0106x; 1.0106x over previous
import jax
import jax.numpy as jnp
from jax import lax
from jax.experimental import pallas as pl
from jax.experimental.pallas import tpu as pltpu
from jax.experimental.pallas import tpu_sc as plsc

D_VOCAB = 1_000_000
D_MODEL = 64
BATCH = 16384

NC = 2
NS = 16
LANES = 16
NW = NC * NS

RPW = BATCH // NW
RPC = 256
NCHUNK = RPW // RPC
IPC = 2 * RPC
ELEMS = IPC * D_MODEL
GROW = 128
NGROW = ELEMS // GROW
FIRE = 8


def _body(xf_hbm, wf_hbm, out_hbm, idx_v, offs_v, g_v, o_v, sem):
    wid = lax.axis_index("s") * NC + lax.axis_index("c")
    dvecs = [
        lax.iota(jnp.int32, LANES) * D_VOCAB + q * LANES * D_VOCAB
        for q in range(D_MODEL // LANES)
    ]

    for c in range(NCHUNK):
        idx_base = wid * (2 * RPW) + c * IPC
        pltpu.sync_copy(xf_hbm.at[pl.ds(idx_base, IPC)], idx_v)

        def build(v, carry):
            idx16 = idx_v[pl.ds(v * LANES, LANES)]
            for l in range(LANES):
                i = idx16[l]
                base = (v * LANES + l) * D_MODEL
                for q in range(D_MODEL // LANES):
                    offs_v[pl.ds(base + q * LANES, LANES)] = dvecs[q] + i
            return carry

        lax.fori_loop(0, IPC // LANES, build, 0)

        pltpu.async_copy(wf_hbm.at[offs_v], g_v, sem).wait()

        def psum(b, carry):
            gb = b * (2 * D_MODEL)
            ob = b * D_MODEL
            for q in range(D_MODEL // LANES):
                o_v[pl.ds(ob + q * LANES, LANES)] = (
                    g_v[pl.ds(gb + q * LANES, LANES)]
                    + g_v[pl.ds(gb + D_MODEL + q * LANES, LANES)]
                )
            return carry

        lax.fori_loop(0, RPC, psum, 0)

        out_base = (wid * RPW + c * RPC) * D_MODEL
        pltpu.sync_copy(o_v, out_hbm.at[pl.ds(out_base, RPC * D_MODEL)])


def _make_sc_call():
    mesh = plsc.VectorSubcoreMesh(
        core_axis_name="c", subcore_axis_name="s", num_cores=NC, num_subcores=NS
    )
    return pl.kernel(
        _body,
        out_type=jax.ShapeDtypeStruct((BATCH * D_MODEL,), jnp.float32),
        mesh=mesh,
        scratch_types=[
            pltpu.VMEM((IPC,), jnp.int32),
            pltpu.VMEM((ELEMS,), jnp.int32),
            pltpu.VMEM((ELEMS,), jnp.float32),
            pltpu.VMEM((RPC * D_MODEL,), jnp.float32),
            pltpu.SemaphoreType.DMA,
        ],
    )


def kernel(x, W_E):
    xf = x.reshape(-1).astype(jnp.int32)
    wf = W_E.reshape(-1)
    out = _make_sc_call()(xf, wf)
    return out.reshape(BATCH, 1, D_MODEL)


# speedup vs baseline: 16.0390x; 15.8713x over previous
import jax
import jax.numpy as jnp
from jax import lax
from jax.experimental import pallas as pl
from jax.experimental.pallas import tpu as pltpu
from jax.experimental.pallas import tpu_sc as plsc

D_VOCAB = 1_000_000
D_MODEL = 64
BATCH = 16384

NC = 2
NS = 16
LANES = 16
NW = NC * NS

RPW = BATCH // NW
IPW = 2 * RPW
RPC = 256
NCHUNK = RPW // RPC
SPC = 2 * RPC

VBH = 4096
NB2 = -(-D_VOCAB // (2 * VBH))
TROWS = NB2 * VBH


def _tbody(a_ref, b_ref, o_ref):
    eye = jnp.eye(2 * D_MODEL, dtype=jnp.float32)
    c = jnp.concatenate([a_ref[...], b_ref[...]], axis=0)
    o_ref[...] = lax.dot_general(
        c, eye, (((0,), (0,)), ((), ())),
        preferred_element_type=jnp.float32,
        precision=lax.Precision.HIGHEST,
    )


def _transpose_call(w):
    return pl.pallas_call(
        _tbody,
        grid=(NB2,),
        in_specs=[
            pl.BlockSpec((D_MODEL, VBH), lambda i: (0, 2 * i)),
            pl.BlockSpec(
                (D_MODEL, VBH),
                lambda i: (0, jnp.minimum(2 * i + 1, -(-D_VOCAB // VBH) - 1)),
            ),
        ],
        out_specs=pl.BlockSpec((VBH, 2 * D_MODEL), lambda i: (i, 0)),
        out_shape=jax.ShapeDtypeStruct((TROWS, 2 * D_MODEL), jnp.float32),
    )(w, w)


def _gbody(xf_hbm, wt_hbm, out_hbm, idx_v, slab_v, g_v, o_v, sem):
    wid = lax.axis_index("s") * NC + lax.axis_index("c")
    pltpu.sync_copy(xf_hbm.at[pl.ds(wid * IPW, IPW)], idx_v)

    def shift(v, carry):
        sl = pl.ds(v * LANES, LANES)
        i = idx_v[sl]
        slab_v[sl] = lax.shift_left(lax.shift_right_logical(i, 13), 12) | (i & (VBH - 1))
        return carry

    lax.fori_loop(0, IPW // LANES, shift, 0)

    for c in range(NCHUNK):
        pltpu.async_copy(
            wt_hbm.at[slab_v.at[pl.ds(c * SPC, SPC)]], g_v, sem
        ).wait()

        def psum(g, carry):
            jbase = g * 2 * LANES
            vA = lax.shift_right_logical(idx_v[pl.ds(c * SPC + jbase, LANES)], 12) & 1
            vB = lax.shift_right_logical(idx_v[pl.ds(c * SPC + jbase + LANES, LANES)], 12) & 1
            for l in range(LANES):
                if l < 8:
                    p0 = vA[2 * l] * D_MODEL
                    p1 = vA[2 * l + 1] * D_MODEL
                else:
                    p0 = vB[2 * l - 16] * D_MODEL
                    p1 = vB[2 * l - 15] * D_MODEL
                b = g * LANES + l
                for q in range(D_MODEL // LANES):
                    o_v[b, pl.ds(q * LANES, LANES)] = (
                        g_v[2 * b, pl.ds(p0 + q * LANES, LANES)]
                        + g_v[2 * b + 1, pl.ds(p1 + q * LANES, LANES)]
                    )
            return carry

        lax.fori_loop(0, RPC // LANES, psum, 0)
        pltpu.sync_copy(o_v, out_hbm.at[pl.ds(wid * RPW + c * RPC, RPC)])


def _gather_call():
    mesh = plsc.VectorSubcoreMesh(
        core_axis_name="c", subcore_axis_name="s", num_cores=NC, num_subcores=NS
    )
    return pl.kernel(
        _gbody,
        out_type=jax.ShapeDtypeStruct((BATCH, D_MODEL), jnp.float32),
        mesh=mesh,
        scratch_types=[
            pltpu.VMEM((IPW,), jnp.int32),
            pltpu.VMEM((IPW,), jnp.int32),
            pltpu.VMEM((SPC, 2 * D_MODEL), jnp.float32),
            pltpu.VMEM((RPC, D_MODEL), jnp.float32),
            pltpu.SemaphoreType.DMA,
        ],
    )


def kernel(x, W_E):
    xf = x.reshape(-1).astype(jnp.int32)
    wt = _transpose_call(W_E)
    out = _gather_call()(xf, wt)
    return out.reshape(BATCH, 1, D_MODEL)


# speedup vs baseline: 20.7305x; 1.2925x over previous
import jax
import jax.numpy as jnp
from jax import lax
from jax.experimental import pallas as pl
from jax.experimental.pallas import tpu as pltpu
from jax.experimental.pallas import tpu_sc as plsc

D_VOCAB = 1_000_000
D_MODEL = 64
BATCH = 16384

NC = 2
NS = 16
LANES = 16
NW = NC * NS

RPW = BATCH // NW
IPW = 2 * RPW
RPC = 256
NCHUNK = RPW // RPC
SPC = 2 * RPC

VBH = 4096
NB2 = -(-D_VOCAB // (2 * VBH))
TROWS = NB2 * VBH


def _tbody(a_ref, b_ref, o_ref):
    c = jnp.concatenate([a_ref[...], b_ref[...]], axis=0)
    o_ref[...] = c.T


def _transpose_call(w):
    return pl.pallas_call(
        _tbody,
        grid=(NB2,),
        in_specs=[
            pl.BlockSpec((D_MODEL, VBH), lambda i: (0, 2 * i)),
            pl.BlockSpec(
                (D_MODEL, VBH),
                lambda i: (0, jnp.minimum(2 * i + 1, -(-D_VOCAB // VBH) - 1)),
            ),
        ],
        out_specs=pl.BlockSpec((VBH, 2 * D_MODEL), lambda i: (i, 0)),
        out_shape=jax.ShapeDtypeStruct((TROWS, 2 * D_MODEL), jnp.float32),
    )(w, w)


def _gbody(xf_hbm, wt_hbm, out_hbm, idx_v, slab_v, g_v, o_v, sem):
    wid = lax.axis_index("s") * NC + lax.axis_index("c")
    pltpu.sync_copy(xf_hbm.at[pl.ds(wid * IPW, IPW)], idx_v)

    def shift(v, carry):
        sl = pl.ds(v * LANES, LANES)
        i = idx_v[sl]
        slab_v[sl] = lax.shift_left(lax.shift_right_logical(i, 13), 12) | (i & (VBH - 1))
        return carry

    lax.fori_loop(0, IPW // LANES, shift, 0)

    for c in range(NCHUNK):
        pltpu.async_copy(
            wt_hbm.at[slab_v.at[pl.ds(c * SPC, SPC)]], g_v, sem
        ).wait()

        def psum(g, carry):
            jbase = g * 2 * LANES
            vA = lax.shift_right_logical(idx_v[pl.ds(c * SPC + jbase, LANES)], 12) & 1
            vB = lax.shift_right_logical(idx_v[pl.ds(c * SPC + jbase + LANES, LANES)], 12) & 1
            for l in range(LANES):
                if l < 8:
                    p0 = vA[2 * l] * D_MODEL
                    p1 = vA[2 * l + 1] * D_MODEL
                else:
                    p0 = vB[2 * l - 16] * D_MODEL
                    p1 = vB[2 * l - 15] * D_MODEL
                b = g * LANES + l
                for q in range(D_MODEL // LANES):
                    o_v[b, pl.ds(q * LANES, LANES)] = (
                        g_v[2 * b, pl.ds(p0 + q * LANES, LANES)]
                        + g_v[2 * b + 1, pl.ds(p1 + q * LANES, LANES)]
                    )
            return carry

        lax.fori_loop(0, RPC // LANES, psum, 0)
        pltpu.sync_copy(o_v, out_hbm.at[pl.ds(wid * RPW + c * RPC, RPC)])


def _gather_call():
    mesh = plsc.VectorSubcoreMesh(
        core_axis_name="c", subcore_axis_name="s", num_cores=NC, num_subcores=NS
    )
    return pl.kernel(
        _gbody,
        out_type=jax.ShapeDtypeStruct((BATCH, D_MODEL), jnp.float32),
        mesh=mesh,
        scratch_types=[
            pltpu.VMEM((IPW,), jnp.int32),
            pltpu.VMEM((IPW,), jnp.int32),
            pltpu.VMEM((SPC, 2 * D_MODEL), jnp.float32),
            pltpu.VMEM((RPC, D_MODEL), jnp.float32),
            pltpu.SemaphoreType.DMA,
        ],
    )


def kernel(x, W_E):
    xf = x.reshape(-1).astype(jnp.int32)
    wt = _transpose_call(W_E)
    out = _gather_call()(xf, wt)
    return out.reshape(BATCH, 1, D_MODEL)
